# baseline (speedup 1.0000x reference)
import functools

import jax
import jax.numpy as jnp
from jax import lax
from jax.experimental import pallas as pl
from jax.experimental.pallas import tpu as pltpu
from jax.experimental.pallas import tpu_sc as plsc

LANES = 16
NSC = 2
NT = 16
NW = NSC * NT
CHW = 32
NA_BLK = 64
WR = 16
SEGC = 64
MAXE = SEGC * CHW
NBB = 176


def _mlp_body(q_ref, w1_ref, b1_ref, w2_ref, b2_ref, o1_ref, o2_ref, o3_ref):
    h = jnp.dot(q_ref[...], w1_ref[...], preferred_element_type=jnp.float32)
    h = h + b1_ref[...]
    h = h * jax.nn.sigmoid(h)
    o = jnp.dot(h, w2_ref[...], preferred_element_type=jnp.float32)
    o = o + b2_ref[...]
    f = q_ref.shape[1]
    o1_ref[...] = o[:, :f]
    o2_ref[...] = o[:, f:2 * f]
    o3_ref[...] = o[:, 2 * f:]


def _mlp(q2, W1, b1, W2, b2):
    n, f = q2.shape
    rt = 1000
    return pl.pallas_call(
        _mlp_body,
        grid=(n // rt,),
        in_specs=[
            pl.BlockSpec((rt, f), lambda i: (i, 0)),
            pl.BlockSpec((f, f), lambda i: (0, 0)),
            pl.BlockSpec((1, f), lambda i: (0, 0)),
            pl.BlockSpec((f, 3 * f), lambda i: (0, 0)),
            pl.BlockSpec((1, 3 * f), lambda i: (0, 0)),
        ],
        out_specs=[pl.BlockSpec((rt, f), lambda i: (i, 0))] * 3,
        out_shape=[jax.ShapeDtypeStruct((n, f), jnp.float32)] * 3,
    )(q2, W1, b1.reshape(1, f), W2, b2.reshape(1, 3 * f))


def _mu_split_body(m_ref, o1_ref, o2_ref, o3_ref):
    o1_ref[...] = m_ref[:, 0, :]
    o2_ref[...] = m_ref[:, 1, :]
    o3_ref[...] = m_ref[:, 2, :]


def _mu_split(mu):
    n, _, f = mu.shape
    rt = 1000
    return pl.pallas_call(
        _mu_split_body,
        grid=(n // rt,),
        in_specs=[pl.BlockSpec((rt, 3, f), lambda i: (i, 0, 0))],
        out_specs=[pl.BlockSpec((rt, f), lambda i: (i, 0))] * 3,
        out_shape=[jax.ShapeDtypeStruct((n, f), jnp.float32)] * 3,
    )(mu)


def _assemble_body(q_ref, o1_ref, o2_ref, o3_ref, qo_ref, mo_ref):
    qo_ref[...] = q_ref[...][:, None, :]
    mo_ref[:, 0, :] = o1_ref[...]
    mo_ref[:, 1, :] = o2_ref[...]
    mo_ref[:, 2, :] = o3_ref[...]


def _assemble(oq, o1, o2, o3):
    n, f = oq.shape
    rt = 1000
    return pl.pallas_call(
        _assemble_body,
        grid=(n // rt,),
        in_specs=[pl.BlockSpec((rt, f), lambda i: (i, 0))] * 4,
        out_specs=[pl.BlockSpec((rt, 1, f), lambda i: (i, 0, 0)),
                   pl.BlockSpec((rt, 3, f), lambda i: (i, 0, 0))],
        out_shape=[jax.ShapeDtypeStruct((n, 1, f), jnp.float32),
                   jax.ShapeDtypeStruct((n, 3, f), jnp.float32)],
    )(oq, o1, o2, o3)


def _splat(vec, lane):
    return jnp.take_along_axis(vec, jnp.full((LANES,), lane, jnp.int32),
                               axis=0)


def _make_sc_kernel(E, F, N, NBLK):
    F3 = 3 * F
    F4 = 4 * F
    BPW = NBLK // NW
    mesh = plsc.VectorSubcoreMesh(core_axis_name="c", subcore_axis_name="s")

    @functools.partial(
        pl.kernel,
        out_type=tuple(jax.ShapeDtypeStruct((N, F), jnp.float32)
                       for _ in range(4)),
        mesh=mesh,
        compiler_params=pltpu.CompilerParams(needs_layout_passes=False),
        scratch_types=[
            pltpu.VMEM((NBB,), jnp.int32),
            pltpu.VMEM((MAXE,), jnp.int32),
            pltpu.VMEM((MAXE,), jnp.int32),
            pltpu.VMEM((MAXE * 3,), jnp.float32),
            pltpu.VMEM((CHW, F3), jnp.float32),
            pltpu.VMEM((CHW, F3), jnp.float32),
            pltpu.VMEM((CHW, F), jnp.float32),
            pltpu.VMEM((CHW, F), jnp.float32),
            pltpu.VMEM((CHW, F), jnp.float32),
            pltpu.VMEM((CHW, F), jnp.float32),
            pltpu.VMEM((CHW, F), jnp.float32),
            pltpu.VMEM((CHW, F), jnp.float32),
            pltpu.VMEM((CHW, F), jnp.float32),
            pltpu.VMEM((CHW, F), jnp.float32),
            pltpu.VMEM((CHW, F), jnp.float32),
            pltpu.VMEM((CHW, F), jnp.float32),
            pltpu.VMEM((CHW, F), jnp.float32),
            pltpu.VMEM((CHW, F), jnp.float32),
            pltpu.VMEM((NA_BLK * F4,), jnp.float32),
            pltpu.VMEM((WR, F), jnp.float32),
            pltpu.VMEM((WR, F), jnp.float32),
            pltpu.VMEM((WR, F), jnp.float32),
            pltpu.VMEM((WR, F), jnp.float32),
            pltpu.SemaphoreType.DMA,
            pltpu.SemaphoreType.DMA,
            pltpu.SemaphoreType.DMA,
            pltpu.SemaphoreType.DMA,
            pltpu.SemaphoreType.DMA,
        ],
    )
    def sc_edges(x1_hbm, x2_hbm, x3_hbm, m1_hbm, m2_hbm, m3_hbm, wij_hbm,
                 dir_hbm, ii_hbm, ij_hbm, bnd_hbm, q_hbm,
                 outq_hbm, o1_hbm, o2_hbm, o3_hbm,
                 bnd_v, ii_sv, ij_sv, dir_sv, wij_0, wij_1,
                 x1_0, x2_0, x3_0, m1_0, m2_0, m3_0,
                 x1_1, x2_1, x3_1, m1_1, m2_1, m3_1,
                 acc_v, stq_v, st1_v, st2_v, st3_v,
                 sem_seg, sem_w0, sem_w1, sem_g0, sem_g1):
        c = lax.axis_index("c")
        t = lax.axis_index("s")
        w = c * NT + t
        lanes = lax.iota(jnp.int32, LANES)
        zero16 = jnp.zeros((LANES,), jnp.float32)
        minval = jnp.int32(-(2 ** 31))
        pltpu.sync_copy(bnd_hbm, bnd_v)
        tabs = (x1_hbm, x2_hbm, x3_hbm, m1_hbm, m2_hbm, m3_hbm)
        bufs = (((x1_0, x2_0, x3_0, m1_0, m2_0, m3_0), wij_0, sem_w0, sem_g0),
                ((x1_1, x2_1, x3_1, m1_1, m2_1, m3_1), wij_1, sem_w1, sem_g1))
        outs = (o1_hbm, o2_hbm, o3_hbm)
        stms = (st1_v, st2_v, st3_v)

        def extract(pos):
            sub = bnd_v[pl.ds((pos // LANES) * LANES, LANES)]
            return jnp.max(jnp.where(lanes == pos % LANES, sub, minval))

        def issue_chunk(ch, fs, p):
            rows, wb, sw, sg = bufs[p]
            off = ch * CHW - fs
            pltpu.async_copy(wij_hbm.at[pl.ds(ch * CHW, CHW)], wb, sw)
            idxr = ij_sv.at[pl.ds(off, CHW)]
            for tab, buf in zip(tabs, rows):
                pltpu.async_copy(tab.at[idxr], buf, sg)

        def wait_chunk(ch, fs, p):
            rows, wb, sw, sg = bufs[p]
            off = ch * CHW - fs
            pltpu.make_async_copy(
                wij_hbm.at[pl.ds(ch * CHW, CHW)], wb, sw).wait()
            idxr = ij_sv.at[pl.ds(off, CHW)]
            for tab, buf in zip(tabs, rows):
                pltpu.make_async_copy(tab.at[idxr], buf, sg).wait()

        def compute_chunk(ch, fs, e0, e1, base, p):
            (x1b, x2b, x3b, m1b, m2b, m3b), wb, _, _ = bufs[p]
            mbs = (m1b, m2b, m3b)
            es = ch * CHW
            eo = es - fs

            def edge_body(e, _):
                pos = es + e

                @pl.when((pos >= e0) & (pos < e1))
                def _():
                    sub = ii_sv[pl.ds(eo + (e // LANES) * LANES, LANES)]
                    rel = _splat(sub, e % LANES) - base
                    rowb = rel * F4
                    for k in range(F // LANES):
                        sl = pl.ds(k * LANES, LANES)
                        v = wb[e, sl] * x1b[e, sl]
                        plsc.addupdate_scatter(
                            acc_v, [rowb + (k * LANES + lanes)], v)
                    e3 = (eo + e) * 3
                    dirs = []
                    for d in range(3):
                        off = e3 + d
                        dsub = dir_sv[pl.ds((off // LANES) * LANES, LANES)]
                        dirs.append(_splat(dsub, off % LANES))
                    for k in range(F // LANES):
                        sl = pl.ds(k * LANES, LANES)
                        sR = pl.ds(F + k * LANES, LANES)
                        sM = pl.ds(2 * F + k * LANES, LANES)
                        wr = wb[e, sR] * x2b[e, sl]
                        wm = wb[e, sM] * x3b[e, sl]
                        for d in range(3):
                            col = (d + 1) * F + k * LANES
                            v = wr * dirs[d] + wm * mbs[d][e, sl]
                            plsc.addupdate_scatter(
                                acc_v, [rowb + (col + lanes)], v)
                return 0

            lax.fori_loop(0, CHW, edge_body, 0)

        def block_body(j, _):
            b = w + NW * j
            base = b * NA_BLK
            e0 = extract(b)
            e1 = extract(b + 1)

            def zero_body(r, _):
                for k in range(F4 // LANES):
                    acc_v[pl.ds(r * F4 + k * LANES, LANES)] = zero16
                return 0

            lax.fori_loop(0, NA_BLK, zero_body, 0)

            c0 = e0 // CHW
            c1 = (e1 + CHW - 1) // CHW
            nseg = (c1 - c0 + SEGC - 1) // SEGC

            def seg_body(s, _):
                cs = c0 + s * SEGC
                ce = jnp.minimum(cs + SEGC, c1)
                fs = jnp.minimum(cs * CHW, E - MAXE)
                s1 = pltpu.async_copy(
                    ii_hbm.at[pl.ds(fs, MAXE)], ii_sv, sem_seg)
                s2 = pltpu.async_copy(
                    ij_hbm.at[pl.ds(fs, MAXE)], ij_sv, sem_seg)
                s3 = pltpu.async_copy(
                    dir_hbm.at[pl.ds(fs * 3, MAXE * 3)], dir_sv, sem_seg)
                s1.wait()
                s2.wait()
                s3.wait()
                issue_chunk(cs, fs, 0)

                def pipe_body(k, _):
                    ch0 = cs + 2 * k
                    ch1 = ch0 + 1
                    wait_chunk(ch0, fs, 0)

                    @pl.when(ch1 < ce)
                    def _():
                        issue_chunk(ch1, fs, 1)

                    compute_chunk(ch0, fs, e0, e1, base, 0)

                    @pl.when(ch1 < ce)
                    def _():
                        wait_chunk(ch1, fs, 1)

                        @pl.when(ch1 + 1 < ce)
                        def _():
                            issue_chunk(ch1 + 1, fs, 0)

                        compute_chunk(ch1, fs, e0, e1, base, 1)
                    return 0

                lax.fori_loop(0, (ce - cs + 1) // 2, pipe_body, 0)
                return 0

            lax.fori_loop(0, nseg, seg_body, 0)

            for s2 in range(NA_BLK // WR):
                r0 = base + s2 * WR

                @pl.when(r0 + WR <= N)
                def _():
                    pltpu.sync_copy(q_hbm.at[pl.ds(r0, WR)], stq_v)
                    for d in range(3):
                        pltpu.sync_copy(
                            tabs[3 + d].at[pl.ds(r0, WR)], stms[d])

                    def add_body(r, _):
                        rb = (s2 * WR + r) * F4
                        for k in range(F // LANES):
                            sl = pl.ds(k * LANES, LANES)
                            a = acc_v[pl.ds(rb + k * LANES, LANES)]
                            stq_v[r, sl] = stq_v[r, sl] + a
                            for d in range(3):
                                a = acc_v[pl.ds(rb + (d + 1) * F + k * LANES,
                                                LANES)]
                                stms[d][r, sl] = stms[d][r, sl] + a
                        return 0

                    lax.fori_loop(0, WR, add_body, 0)
                    pltpu.sync_copy(stq_v, outq_hbm.at[pl.ds(r0, WR)])
                    for d in range(3):
                        pltpu.sync_copy(stms[d], outs[d].at[pl.ds(r0, WR)])
            return 0

        lax.fori_loop(0, BPW, block_body, 0)

    return sc_edges


def kernel(q, mu, Wij, dir_ij, W1, b1, W2, b2, idx_i, idx_j, n_atoms):
    N = q.shape[0]
    F = q.shape[-1]
    E = Wij.shape[0]
    NBLK = -(-N // NA_BLK)
    NBLK = -(-NBLK // NW) * NW

    idx_i32 = idx_i.astype(jnp.int32)
    idx_j32 = idx_j.astype(jnp.int32)
    q2 = q.reshape(N, F)
    x1, x2, x3 = _mlp(q2, W1, b1, W2, b2)
    m1, m2, m3 = _mu_split(mu)
    bounds = jnp.searchsorted(
        idx_i32, (jnp.arange(NBLK + 1) * NA_BLK).astype(jnp.int32)
    ).astype(jnp.int32)
    bounds = jnp.pad(bounds, (0, NBB - (NBLK + 1)))

    sc_edges = _make_sc_kernel(E, F, N, NBLK)
    oq, o1, o2, o3 = sc_edges(x1, x2, x3, m1, m2, m3, Wij.reshape(E, 3 * F),
                              dir_ij.reshape(E * 3), idx_i32, idx_j32,
                              bounds, q2)
    return tuple(_assemble(oq, o1, o2, o3))


# speedup vs baseline: 1.0913x; 1.0913x over previous
import functools

import jax
import jax.numpy as jnp
from jax import lax
from jax.experimental import pallas as pl
from jax.experimental.pallas import tpu as pltpu
from jax.experimental.pallas import tpu_sc as plsc

LANES = 16
NSC = 2
NT = 16
NW = NSC * NT
CHW = 32
NA_BLK = 64
WR = 16
SEGC = 64
MAXE = SEGC * CHW
NBB = 176


def _mlp_body(q_ref, w1_ref, b1_ref, w2_ref, b2_ref, o1_ref, o2_ref, o3_ref):
    h = jnp.dot(q_ref[...], w1_ref[...], preferred_element_type=jnp.float32)
    h = h + b1_ref[...]
    h = h * jax.nn.sigmoid(h)
    o = jnp.dot(h, w2_ref[...], preferred_element_type=jnp.float32)
    o = o + b2_ref[...]
    f = q_ref.shape[1]
    o1_ref[...] = o[:, :f]
    o2_ref[...] = o[:, f:2 * f]
    o3_ref[...] = o[:, 2 * f:]


def _mlp(q2, W1, b1, W2, b2):
    n, f = q2.shape
    rt = 1000
    return pl.pallas_call(
        _mlp_body,
        grid=(n // rt,),
        in_specs=[
            pl.BlockSpec((rt, f), lambda i: (i, 0)),
            pl.BlockSpec((f, f), lambda i: (0, 0)),
            pl.BlockSpec((1, f), lambda i: (0, 0)),
            pl.BlockSpec((f, 3 * f), lambda i: (0, 0)),
            pl.BlockSpec((1, 3 * f), lambda i: (0, 0)),
        ],
        out_specs=[pl.BlockSpec((rt, f), lambda i: (i, 0))] * 3,
        out_shape=[jax.ShapeDtypeStruct((n, f), jnp.float32)] * 3,
    )(q2, W1, b1.reshape(1, f), W2, b2.reshape(1, 3 * f))


def _mu_split_body(m_ref, o1_ref, o2_ref, o3_ref):
    o1_ref[...] = m_ref[:, 0, :]
    o2_ref[...] = m_ref[:, 1, :]
    o3_ref[...] = m_ref[:, 2, :]


def _mu_split(mu):
    n, _, f = mu.shape
    rt = 1000
    return pl.pallas_call(
        _mu_split_body,
        grid=(n // rt,),
        in_specs=[pl.BlockSpec((rt, 3, f), lambda i: (i, 0, 0))],
        out_specs=[pl.BlockSpec((rt, f), lambda i: (i, 0))] * 3,
        out_shape=[jax.ShapeDtypeStruct((n, f), jnp.float32)] * 3,
    )(mu)


def _assemble_body(q_ref, o1_ref, o2_ref, o3_ref, qo_ref, mo_ref):
    qo_ref[...] = q_ref[...][:, None, :]
    mo_ref[:, 0, :] = o1_ref[...]
    mo_ref[:, 1, :] = o2_ref[...]
    mo_ref[:, 2, :] = o3_ref[...]


def _assemble(oq, o1, o2, o3):
    n, f = oq.shape
    rt = 1000
    return pl.pallas_call(
        _assemble_body,
        grid=(n // rt,),
        in_specs=[pl.BlockSpec((rt, f), lambda i: (i, 0))] * 4,
        out_specs=[pl.BlockSpec((rt, 1, f), lambda i: (i, 0, 0)),
                   pl.BlockSpec((rt, 3, f), lambda i: (i, 0, 0))],
        out_shape=[jax.ShapeDtypeStruct((n, 1, f), jnp.float32),
                   jax.ShapeDtypeStruct((n, 3, f), jnp.float32)],
    )(oq, o1, o2, o3)


def _splat(vec, lane):
    return jnp.take_along_axis(vec, jnp.full((LANES,), lane, jnp.int32),
                               axis=0)


def _make_sc_kernel(E, F, N, NBLK):
    F3 = 3 * F
    F4 = 4 * F
    BPW = NBLK // NW
    mesh = plsc.VectorSubcoreMesh(core_axis_name="c", subcore_axis_name="s")

    @functools.partial(
        pl.kernel,
        out_type=tuple(jax.ShapeDtypeStruct((N, F), jnp.float32)
                       for _ in range(4)),
        mesh=mesh,
        compiler_params=pltpu.CompilerParams(needs_layout_passes=False),
        scratch_types=[
            pltpu.VMEM((NBB,), jnp.int32),
            pltpu.VMEM((MAXE,), jnp.int32),
            pltpu.VMEM((MAXE,), jnp.int32),
            pltpu.VMEM((MAXE * 3,), jnp.float32),
            pltpu.VMEM((CHW, F3), jnp.float32),
            pltpu.VMEM((CHW, F3), jnp.float32),
            pltpu.VMEM((CHW, F), jnp.float32),
            pltpu.VMEM((CHW, F), jnp.float32),
            pltpu.VMEM((CHW, F), jnp.float32),
            pltpu.VMEM((CHW, F), jnp.float32),
            pltpu.VMEM((CHW, F), jnp.float32),
            pltpu.VMEM((CHW, F), jnp.float32),
            pltpu.VMEM((CHW, F), jnp.float32),
            pltpu.VMEM((CHW, F), jnp.float32),
            pltpu.VMEM((CHW, F), jnp.float32),
            pltpu.VMEM((CHW, F), jnp.float32),
            pltpu.VMEM((CHW, F), jnp.float32),
            pltpu.VMEM((CHW, F), jnp.float32),
            pltpu.VMEM((NA_BLK * F4,), jnp.float32),
            pltpu.VMEM((WR, F), jnp.float32),
            pltpu.VMEM((WR, F), jnp.float32),
            pltpu.VMEM((WR, F), jnp.float32),
            pltpu.VMEM((WR, F), jnp.float32),
            pltpu.SemaphoreType.DMA,
            pltpu.SemaphoreType.DMA,
            pltpu.SemaphoreType.DMA,
            pltpu.SemaphoreType.DMA,
            pltpu.SemaphoreType.DMA,
        ],
    )
    def sc_edges(x1_hbm, x2_hbm, x3_hbm, m1_hbm, m2_hbm, m3_hbm, wij_hbm,
                 dir_hbm, ii_hbm, ij_hbm, bnd_hbm, q_hbm,
                 outq_hbm, o1_hbm, o2_hbm, o3_hbm,
                 bnd_v, ii_sv, ij_sv, dir_sv, wij_0, wij_1,
                 x1_0, x2_0, x3_0, m1_0, m2_0, m3_0,
                 x1_1, x2_1, x3_1, m1_1, m2_1, m3_1,
                 acc_v, stq_v, st1_v, st2_v, st3_v,
                 sem_seg, sem_w0, sem_w1, sem_g0, sem_g1):
        c = lax.axis_index("c")
        t = lax.axis_index("s")
        w = c * NT + t
        lanes = lax.iota(jnp.int32, LANES)
        zero16 = jnp.zeros((LANES,), jnp.float32)
        minval = jnp.int32(-(2 ** 31))
        pltpu.sync_copy(bnd_hbm, bnd_v)
        tabs = (x1_hbm, x2_hbm, x3_hbm, m1_hbm, m2_hbm, m3_hbm)
        bufs = (((x1_0, x2_0, x3_0, m1_0, m2_0, m3_0), wij_0, sem_w0, sem_g0),
                ((x1_1, x2_1, x3_1, m1_1, m2_1, m3_1), wij_1, sem_w1, sem_g1))
        outs = (o1_hbm, o2_hbm, o3_hbm)
        stms = (st1_v, st2_v, st3_v)

        def extract(pos):
            sub = bnd_v[pl.ds((pos // LANES) * LANES, LANES)]
            return jnp.max(jnp.where(lanes == pos % LANES, sub, minval))

        def issue_chunk(ch, fs, p):
            rows, wb, sw, sg = bufs[p]
            off = ch * CHW - fs
            pltpu.async_copy(wij_hbm.at[pl.ds(ch * CHW, CHW), 0], wb, sw)
            idxr = ij_sv.at[pl.ds(off, CHW)]
            for tab, buf in zip(tabs, rows):
                pltpu.async_copy(tab.at[idxr], buf, sg)

        def wait_chunk(ch, fs, p):
            rows, wb, sw, sg = bufs[p]
            off = ch * CHW - fs
            pltpu.make_async_copy(
                wij_hbm.at[pl.ds(ch * CHW, CHW), 0], wb, sw).wait()
            idxr = ij_sv.at[pl.ds(off, CHW)]
            for tab, buf in zip(tabs, rows):
                pltpu.make_async_copy(tab.at[idxr], buf, sg).wait()

        def compute_chunk(ch, fs, e0, e1, base, p):
            (x1b, x2b, x3b, m1b, m2b, m3b), wb, _, _ = bufs[p]
            mbs = (m1b, m2b, m3b)
            es = ch * CHW
            eo = es - fs

            def edge_body(e, _):
                pos = es + e

                @pl.when((pos >= e0) & (pos < e1))
                def _():
                    sub = ii_sv[pl.ds(eo + (e // LANES) * LANES, LANES)]
                    rel = _splat(sub, e % LANES) - base
                    rowb = rel * F4
                    for k in range(F // LANES):
                        sl = pl.ds(k * LANES, LANES)
                        v = wb[e, sl] * x1b[e, sl]
                        plsc.addupdate_scatter(
                            acc_v, [rowb + (k * LANES + lanes)], v)
                    e3 = (eo + e) * 3
                    dirs = []
                    for d in range(3):
                        off = e3 + d
                        dsub = dir_sv[pl.ds((off // LANES) * LANES, LANES)]
                        dirs.append(_splat(dsub, off % LANES))
                    for k in range(F // LANES):
                        sl = pl.ds(k * LANES, LANES)
                        sR = pl.ds(F + k * LANES, LANES)
                        sM = pl.ds(2 * F + k * LANES, LANES)
                        wr = wb[e, sR] * x2b[e, sl]
                        wm = wb[e, sM] * x3b[e, sl]
                        for d in range(3):
                            col = (d + 1) * F + k * LANES
                            v = wr * dirs[d] + wm * mbs[d][e, sl]
                            plsc.addupdate_scatter(
                                acc_v, [rowb + (col + lanes)], v)
                return 0

            lax.fori_loop(0, CHW, edge_body, 0)

        def block_body(j, _):
            b = w + NW * j
            base = b * NA_BLK
            e0 = extract(b)
            e1 = extract(b + 1)

            def zero_body(r, _):
                for k in range(F4 // LANES):
                    acc_v[pl.ds(r * F4 + k * LANES, LANES)] = zero16
                return 0

            lax.fori_loop(0, NA_BLK, zero_body, 0)

            c0 = e0 // CHW
            c1 = (e1 + CHW - 1) // CHW
            nseg = (c1 - c0 + SEGC - 1) // SEGC

            def seg_body(s, _):
                cs = c0 + s * SEGC
                ce = jnp.minimum(cs + SEGC, c1)
                fs = jnp.minimum(cs * CHW, E - MAXE)
                s1 = pltpu.async_copy(
                    ii_hbm.at[pl.ds(fs, MAXE)], ii_sv, sem_seg)
                s2 = pltpu.async_copy(
                    ij_hbm.at[pl.ds(fs, MAXE)], ij_sv, sem_seg)
                s3 = pltpu.async_copy(
                    dir_hbm.at[pl.ds(fs * 3, MAXE * 3)], dir_sv, sem_seg)
                s1.wait()
                s2.wait()
                s3.wait()
                issue_chunk(cs, fs, 0)

                def pipe_body(k, _):
                    ch0 = cs + 2 * k
                    ch1 = ch0 + 1
                    wait_chunk(ch0, fs, 0)

                    @pl.when(ch1 < ce)
                    def _():
                        issue_chunk(ch1, fs, 1)

                    compute_chunk(ch0, fs, e0, e1, base, 0)

                    @pl.when(ch1 < ce)
                    def _():
                        wait_chunk(ch1, fs, 1)

                        @pl.when(ch1 + 1 < ce)
                        def _():
                            issue_chunk(ch1 + 1, fs, 0)

                        compute_chunk(ch1, fs, e0, e1, base, 1)
                    return 0

                lax.fori_loop(0, (ce - cs + 1) // 2, pipe_body, 0)
                return 0

            lax.fori_loop(0, nseg, seg_body, 0)

            for s2 in range(NA_BLK // WR):
                r0 = base + s2 * WR

                @pl.when(r0 + WR <= N)
                def _():
                    pltpu.sync_copy(q_hbm.at[pl.ds(r0, WR)], stq_v)
                    for d in range(3):
                        pltpu.sync_copy(
                            tabs[3 + d].at[pl.ds(r0, WR)], stms[d])

                    def add_body(r, _):
                        rb = (s2 * WR + r) * F4
                        for k in range(F // LANES):
                            sl = pl.ds(k * LANES, LANES)
                            a = acc_v[pl.ds(rb + k * LANES, LANES)]
                            stq_v[r, sl] = stq_v[r, sl] + a
                            for d in range(3):
                                a = acc_v[pl.ds(rb + (d + 1) * F + k * LANES,
                                                LANES)]
                                stms[d][r, sl] = stms[d][r, sl] + a
                        return 0

                    lax.fori_loop(0, WR, add_body, 0)
                    pltpu.sync_copy(stq_v, outq_hbm.at[pl.ds(r0, WR)])
                    for d in range(3):
                        pltpu.sync_copy(stms[d], outs[d].at[pl.ds(r0, WR)])
            return 0

        lax.fori_loop(0, BPW, block_body, 0)

    return sc_edges


def kernel(q, mu, Wij, dir_ij, W1, b1, W2, b2, idx_i, idx_j, n_atoms):
    N = q.shape[0]
    F = q.shape[-1]
    E = Wij.shape[0]
    NBLK = -(-N // NA_BLK)
    NBLK = -(-NBLK // NW) * NW

    idx_i32 = idx_i.astype(jnp.int32)
    idx_j32 = idx_j.astype(jnp.int32)
    q2 = q.reshape(N, F)
    x1, x2, x3 = _mlp(q2, W1, b1, W2, b2)
    m1, m2, m3 = _mu_split(mu)
    bounds = jnp.searchsorted(
        idx_i32, (jnp.arange(NBLK + 1) * NA_BLK).astype(jnp.int32)
    ).astype(jnp.int32)
    bounds = jnp.pad(bounds, (0, NBB - (NBLK + 1)))

    sc_edges = _make_sc_kernel(E, F, N, NBLK)
    oq, o1, o2, o3 = sc_edges(x1, x2, x3, m1, m2, m3, Wij,
                              dir_ij.reshape(E * 3), idx_i32, idx_j32,
                              bounds, q2)
    mu_out = jnp.stack([o1, o2, o3], axis=1)
    return (oq.reshape(N, 1, F), mu_out)
